# initial kernel scaffold (unmeasured)
import jax
import jax.numpy as jnp
from jax import lax
from jax.experimental import pallas as pl
from jax.experimental.pallas import tpu as pltpu

N_DEV = 4
M_PER = 1024
K = 4096
N_PER = 2048
F8_MAX = 448.0

_MESH = pl.DeviceIdType.MESH


def _ag_body(x_hbm, xfull, send_sems, recv_sems, copy_sem):
    me = lax.axis_index("i")
    right = lax.rem(me + 1, N_DEV)
    left = lax.rem(me + N_DEV - 1, N_DEV)

    barrier_sem = pltpu.get_barrier_semaphore()
    pl.semaphore_signal(barrier_sem, inc=1, device_id=(left,), device_id_type=_MESH)
    pl.semaphore_signal(barrier_sem, inc=1, device_id=(right,), device_id_type=_MESH)
    pl.semaphore_wait(barrier_sem, 2)

    cp = pltpu.make_async_copy(x_hbm, xfull.at[pl.ds(me * M_PER, M_PER), :], copy_sem)
    cp.start()

    for h in range(N_DEV - 1):
        o = lax.rem(me - h + N_DEV, N_DEV)
        src = x_hbm if h == 0 else xfull.at[pl.ds(o * M_PER, M_PER), :]
        rdma = pltpu.make_async_remote_copy(
            src_ref=src,
            dst_ref=xfull.at[pl.ds(o * M_PER, M_PER), :],
            send_sem=send_sems.at[h],
            recv_sem=recv_sems.at[h],
            device_id=(right,),
            device_id_type=_MESH,
        )
        rdma.start()
        rdma.wait()
    cp.wait()


def _all_gather(x):
    return pl.pallas_call(
        _ag_body,
        out_shape=jax.ShapeDtypeStruct((N_DEV * M_PER, K), jnp.float32),
        in_specs=[pl.BlockSpec(memory_space=pltpu.ANY)],
        out_specs=pl.BlockSpec(memory_space=pltpu.ANY),
        scratch_shapes=[
            pltpu.SemaphoreType.DMA((N_DEV - 1,)),
            pltpu.SemaphoreType.DMA((N_DEV - 1,)),
            pltpu.SemaphoreType.DMA,
        ],
        compiler_params=pltpu.CompilerParams(collective_id=0),
    )(x)


_BM = 256
_BN = 1024
_GN = N_PER // _BN
_GM = (N_DEV * M_PER) // _BM


def _gemm_body(x_ref, w_ref, y_ref, amax_ref, amax_s):
    n = pl.program_id(0)
    m = pl.program_id(1)

    @pl.when(jnp.logical_and(n == 0, m == 0))
    def _():
        amax_s[0] = 0.0

    yb = jnp.maximum(
        jnp.dot(x_ref[...], w_ref[...], preferred_element_type=jnp.float32), 0.0
    )
    y_ref[...] = yb
    amax_s[0] = jnp.maximum(amax_s[0], jnp.max(yb))

    @pl.when(jnp.logical_and(n == _GN - 1, m == _GM - 1))
    def _():
        amax_ref[0, 0] = amax_s[0]


def _gemm(xfull, w):
    return pl.pallas_call(
        _gemm_body,
        grid=(_GN, _GM),
        out_shape=(
            jax.ShapeDtypeStruct((N_DEV * M_PER, N_PER), jnp.float32),
            jax.ShapeDtypeStruct((1, 1), jnp.float32),
        ),
        in_specs=[
            pl.BlockSpec((_BM, K), lambda n, m: (m, 0)),
            pl.BlockSpec((K, _BN), lambda n, m: (0, n)),
        ],
        out_specs=(
            pl.BlockSpec((_BM, _BN), lambda n, m: (m, n)),
            pl.BlockSpec((1, 1), lambda n, m: (0, 0)),
        ),
        scratch_shapes=[pltpu.SMEM((1,), jnp.float32)],
    )(xfull, w)


_EB = 512
_EG = (N_DEV * M_PER) // _EB


def _epi_body(amax_in, y_ref, o_ref, slots, mybuf, send_sems, recv_sems, ack_sem, scale_s):
    step = pl.program_id(0)

    @pl.when(step == 0)
    def _():
        me = lax.axis_index("i")
        barrier_sem = pltpu.get_barrier_semaphore()
        for k in range(1, N_DEV):
            p = lax.rem(me + k, N_DEV)
            pl.semaphore_signal(barrier_sem, inc=1, device_id=(p,), device_id_type=_MESH)
        pl.semaphore_wait(barrier_sem, N_DEV - 1)

        my_amax = amax_in[0, 0]
        mybuf[...] = jnp.full((8, 128), my_amax, jnp.float32)

        sends = []
        for k in range(1, N_DEV):
            p = lax.rem(me + k, N_DEV)
            s = pltpu.make_async_remote_copy(
                src_ref=mybuf,
                dst_ref=slots.at[me],
                send_sem=send_sems.at[k - 1],
                recv_sem=recv_sems.at[me],
                device_id=(p,),
                device_id_type=_MESH,
            )
            s.start()
            sends.append(s)
        for k in range(1, N_DEV):
            p = lax.rem(me + k, N_DEV)
            r = pltpu.make_async_remote_copy(
                src_ref=mybuf,
                dst_ref=slots.at[p],
                send_sem=send_sems.at[k - 1],
                recv_sem=recv_sems.at[p],
                device_id=(p,),
                device_id_type=_MESH,
            )
            r.wait_recv()
        for s in sends:
            s.wait_send()

        vals = slots[:, 0, :]
        row = lax.broadcasted_iota(jnp.int32, (N_DEV, 128), 0)
        g = jnp.max(jnp.where(row == me, -jnp.inf, vals))
        g = jnp.maximum(g, my_amax)
        scale_s[0] = g / F8_MAX
        scale_s[1] = F8_MAX / g

        for k in range(1, N_DEV):
            p = lax.rem(me + k, N_DEV)
            pl.semaphore_signal(ack_sem, inc=1, device_id=(p,), device_id_type=_MESH)
        pl.semaphore_wait(ack_sem, N_DEV - 1)

    scale = scale_s[0]
    inv = scale_s[1]
    q = (y_ref[...] * inv).astype(jnp.float8_e4m3fn).astype(jnp.float32)
    o_ref[...] = q * scale


def _epilogue(y, amax):
    return pl.pallas_call(
        _epi_body,
        grid=(_EG,),
        out_shape=jax.ShapeDtypeStruct((N_DEV * M_PER, N_PER), jnp.float32),
        in_specs=[
            pl.BlockSpec((1, 1), lambda m: (0, 0)),
            pl.BlockSpec((_EB, N_PER), lambda m: (m, 0)),
        ],
        out_specs=pl.BlockSpec((_EB, N_PER), lambda m: (m, 0)),
        scratch_shapes=[
            pltpu.VMEM((N_DEV, 8, 128), jnp.float32),
            pltpu.VMEM((8, 128), jnp.float32),
            pltpu.SemaphoreType.DMA((N_DEV - 1,)),
            pltpu.SemaphoreType.DMA((N_DEV,)),
            pltpu.SemaphoreType.REGULAR,
            pltpu.SMEM((2,), jnp.float32),
        ],
        compiler_params=pltpu.CompilerParams(collective_id=1),
    )(amax, y)


def kernel(x, w_mat):
    xfull = _all_gather(x)
    y, amax = _gemm(xfull, w_mat)
    return _epilogue(y, amax)


# baseline (device time: 705503 ns/iter reference)
import jax
import jax.numpy as jnp
from jax import lax
from jax.experimental import pallas as pl
from jax.experimental.pallas import tpu as pltpu

N_DEV = 4
M_PER = 1024
K = 4096
N_PER = 2048
F8_MAX = 448.0

_MESH = pl.DeviceIdType.MESH


def _ag_body(x_hbm, xfull, send_sems, recv_sems, copy_sem):
    me = lax.axis_index("i")
    right = lax.rem(me + 1, N_DEV)
    left = lax.rem(me + N_DEV - 1, N_DEV)

    barrier_sem = pltpu.get_barrier_semaphore()
    pl.semaphore_signal(barrier_sem, inc=1, device_id=(left,), device_id_type=_MESH)
    pl.semaphore_signal(barrier_sem, inc=1, device_id=(right,), device_id_type=_MESH)
    pl.semaphore_wait(barrier_sem, 2)

    cp = pltpu.make_async_copy(x_hbm, xfull.at[pl.ds(me * M_PER, M_PER), :], copy_sem)
    cp.start()

    for h in range(N_DEV - 1):
        o = lax.rem(me - h + N_DEV, N_DEV)
        src = x_hbm if h == 0 else xfull.at[pl.ds(o * M_PER, M_PER), :]
        rdma = pltpu.make_async_remote_copy(
            src_ref=src,
            dst_ref=xfull.at[pl.ds(o * M_PER, M_PER), :],
            send_sem=send_sems.at[h],
            recv_sem=recv_sems.at[h],
            device_id=(right,),
            device_id_type=_MESH,
        )
        rdma.start()
        rdma.wait()
    cp.wait()


def _all_gather(x):
    return pl.pallas_call(
        _ag_body,
        out_shape=jax.ShapeDtypeStruct((N_DEV * M_PER, K), jnp.float32),
        in_specs=[pl.BlockSpec(memory_space=pl.ANY)],
        out_specs=pl.BlockSpec(memory_space=pl.ANY),
        scratch_shapes=[
            pltpu.SemaphoreType.DMA((N_DEV - 1,)),
            pltpu.SemaphoreType.DMA((N_DEV - 1,)),
            pltpu.SemaphoreType.DMA,
        ],
        compiler_params=pltpu.CompilerParams(collective_id=0),
    )(x)


_BM = 256
_BN = 1024
_GN = N_PER // _BN
_GM = (N_DEV * M_PER) // _BM


def _gemm_body(x_ref, w_ref, y_ref, amax_ref, amax_s):
    n = pl.program_id(0)
    m = pl.program_id(1)

    @pl.when(jnp.logical_and(n == 0, m == 0))
    def _():
        amax_s[0] = 0.0

    yb = jnp.maximum(
        jnp.dot(x_ref[...], w_ref[...], preferred_element_type=jnp.float32), 0.0
    )
    y_ref[...] = yb
    amax_s[0] = jnp.maximum(amax_s[0], jnp.max(yb))

    @pl.when(jnp.logical_and(n == _GN - 1, m == _GM - 1))
    def _():
        amax_ref[...] = jnp.full((1, 1), amax_s[0], jnp.float32)


def _gemm(xfull, w):
    return pl.pallas_call(
        _gemm_body,
        grid=(_GN, _GM),
        out_shape=(
            jax.ShapeDtypeStruct((N_DEV * M_PER, N_PER), jnp.float32),
            jax.ShapeDtypeStruct((1, 1), jnp.float32),
        ),
        in_specs=[
            pl.BlockSpec((_BM, K), lambda n, m: (m, 0)),
            pl.BlockSpec((K, _BN), lambda n, m: (0, n)),
        ],
        out_specs=(
            pl.BlockSpec((_BM, _BN), lambda n, m: (m, n)),
            pl.BlockSpec((1, 1), lambda n, m: (0, 0)),
        ),
        scratch_shapes=[pltpu.SMEM((1,), jnp.float32)],
        compiler_params=pltpu.CompilerParams(vmem_limit_bytes=60 * 1024 * 1024),
    )(xfull, w)


_EB = 512
_EG = (N_DEV * M_PER) // _EB


def _epi_body(amax_in, y_ref, o_ref, slots, mybuf, send_sems, recv_sems, ack_sem, scale_s):
    step = pl.program_id(0)

    @pl.when(step == 0)
    def _():
        me = lax.axis_index("i")
        barrier_sem = pltpu.get_barrier_semaphore()
        for k in range(1, N_DEV):
            p = lax.rem(me + k, N_DEV)
            pl.semaphore_signal(barrier_sem, inc=1, device_id=(p,), device_id_type=_MESH)
        pl.semaphore_wait(barrier_sem, N_DEV - 1)

        my_amax = amax_in[0, 0]
        mybuf[...] = jnp.full((8, 128), my_amax, jnp.float32)

        sends = []
        for k in range(1, N_DEV):
            p = lax.rem(me + k, N_DEV)
            s = pltpu.make_async_remote_copy(
                src_ref=mybuf,
                dst_ref=slots.at[me],
                send_sem=send_sems.at[k - 1],
                recv_sem=recv_sems.at[me],
                device_id=(p,),
                device_id_type=_MESH,
            )
            s.start()
            sends.append(s)
        for k in range(1, N_DEV):
            p = lax.rem(me + k, N_DEV)
            r = pltpu.make_async_remote_copy(
                src_ref=mybuf,
                dst_ref=slots.at[p],
                send_sem=send_sems.at[k - 1],
                recv_sem=recv_sems.at[p],
                device_id=(p,),
                device_id_type=_MESH,
            )
            r.wait_recv()
        for s in sends:
            s.wait_send()

        vals = slots[:, 0, :]
        row = lax.broadcasted_iota(jnp.int32, (N_DEV, 128), 0)
        g = jnp.max(jnp.where(row == me, -jnp.inf, vals))
        g = jnp.maximum(g, my_amax)
        scale_s[0] = g / F8_MAX
        scale_s[1] = F8_MAX / g

        for k in range(1, N_DEV):
            p = lax.rem(me + k, N_DEV)
            pl.semaphore_signal(ack_sem, inc=1, device_id=(p,), device_id_type=_MESH)
        pl.semaphore_wait(ack_sem, N_DEV - 1)

    scale = scale_s[0]
    inv = scale_s[1]
    q = (y_ref[...] * inv).astype(jnp.float8_e4m3fn).astype(jnp.float32)
    o_ref[...] = q * scale


def _epilogue(y, amax):
    return pl.pallas_call(
        _epi_body,
        grid=(_EG,),
        out_shape=jax.ShapeDtypeStruct((N_DEV * M_PER, N_PER), jnp.float32),
        in_specs=[
            pl.BlockSpec((1, 1), lambda m: (0, 0)),
            pl.BlockSpec((_EB, N_PER), lambda m: (m, 0)),
        ],
        out_specs=pl.BlockSpec((_EB, N_PER), lambda m: (m, 0)),
        scratch_shapes=[
            pltpu.VMEM((N_DEV, 8, 128), jnp.float32),
            pltpu.VMEM((8, 128), jnp.float32),
            pltpu.SemaphoreType.DMA((N_DEV - 1,)),
            pltpu.SemaphoreType.DMA((N_DEV,)),
            pltpu.SemaphoreType.REGULAR,
            pltpu.SMEM((2,), jnp.float32),
        ],
        compiler_params=pltpu.CompilerParams(collective_id=1),
    )(amax, y)


def kernel(x, w_mat):
    xfull = _all_gather(x)
    y, amax = _gemm(xfull, w_mat)
    return _epilogue(y, amax)


# device time: 437192 ns/iter; 1.6137x vs baseline; 1.6137x over previous
import jax
import jax.numpy as jnp
from jax import lax
from jax.experimental import pallas as pl
from jax.experimental.pallas import tpu as pltpu

N_DEV = 4
M_PER = 1024
K = 4096
N_PER = 2048
F8_MAX = 448.0

_MESH = pl.DeviceIdType.MESH


M_HALF = M_PER // 2


def _ag_body(x_hbm, xfull, cw_send, cw_recv, ccw_send, ccw_recv, copy_sem):
    me = lax.axis_index("i")
    right = lax.rem(me + 1, N_DEV)
    left = lax.rem(me + N_DEV - 1, N_DEV)

    barrier_sem = pltpu.get_barrier_semaphore()
    pl.semaphore_signal(barrier_sem, inc=1, device_id=(left,), device_id_type=_MESH)
    pl.semaphore_signal(barrier_sem, inc=1, device_id=(right,), device_id_type=_MESH)
    pl.semaphore_wait(barrier_sem, 2)

    cp = pltpu.make_async_copy(x_hbm, xfull.at[pl.ds(me * M_PER, M_PER), :], copy_sem)
    cp.start()

    for h in range(N_DEV - 1):
        o_cw = lax.rem(me - h + N_DEV, N_DEV)
        o_ccw = lax.rem(me + h, N_DEV)
        src_cw = (
            x_hbm.at[pl.ds(0, M_HALF), :]
            if h == 0
            else xfull.at[pl.ds(o_cw * M_PER, M_HALF), :]
        )
        src_ccw = (
            x_hbm.at[pl.ds(M_HALF, M_HALF), :]
            if h == 0
            else xfull.at[pl.ds(o_ccw * M_PER + M_HALF, M_HALF), :]
        )
        cw = pltpu.make_async_remote_copy(
            src_ref=src_cw,
            dst_ref=xfull.at[pl.ds(o_cw * M_PER, M_HALF), :],
            send_sem=cw_send.at[h],
            recv_sem=cw_recv.at[h],
            device_id=(right,),
            device_id_type=_MESH,
        )
        ccw = pltpu.make_async_remote_copy(
            src_ref=src_ccw,
            dst_ref=xfull.at[pl.ds(o_ccw * M_PER + M_HALF, M_HALF), :],
            send_sem=ccw_send.at[h],
            recv_sem=ccw_recv.at[h],
            device_id=(left,),
            device_id_type=_MESH,
        )
        cw.start()
        ccw.start()
        cw.wait()
        ccw.wait()
    cp.wait()


def _all_gather(x):
    return pl.pallas_call(
        _ag_body,
        out_shape=jax.ShapeDtypeStruct((N_DEV * M_PER, K), jnp.float32),
        in_specs=[pl.BlockSpec(memory_space=pl.ANY)],
        out_specs=pl.BlockSpec(memory_space=pl.ANY),
        scratch_shapes=[
            pltpu.SemaphoreType.DMA((N_DEV - 1,)),
            pltpu.SemaphoreType.DMA((N_DEV - 1,)),
            pltpu.SemaphoreType.DMA((N_DEV - 1,)),
            pltpu.SemaphoreType.DMA((N_DEV - 1,)),
            pltpu.SemaphoreType.DMA,
        ],
        compiler_params=pltpu.CompilerParams(collective_id=0),
    )(x)


_BM = 256
_BN = 1024
_GN = N_PER // _BN
_GM = (N_DEV * M_PER) // _BM


def _gemm_body(x_ref, w_ref, y_ref, amax_ref, amax_s):
    n = pl.program_id(0)
    m = pl.program_id(1)

    @pl.when(jnp.logical_and(n == 0, m == 0))
    def _():
        amax_s[0] = 0.0

    yb = jnp.maximum(
        jnp.dot(x_ref[...], w_ref[...], preferred_element_type=jnp.float32), 0.0
    )
    y_ref[...] = yb
    amax_s[0] = jnp.maximum(amax_s[0], jnp.max(yb))

    @pl.when(jnp.logical_and(n == _GN - 1, m == _GM - 1))
    def _():
        amax_ref[...] = jnp.full((1, 1), amax_s[0], jnp.float32)


def _gemm(xfull, w):
    return pl.pallas_call(
        _gemm_body,
        grid=(_GN, _GM),
        out_shape=(
            jax.ShapeDtypeStruct((N_DEV * M_PER, N_PER), jnp.float32),
            jax.ShapeDtypeStruct((1, 1), jnp.float32),
        ),
        in_specs=[
            pl.BlockSpec((_BM, K), lambda n, m: (m, 0)),
            pl.BlockSpec((K, _BN), lambda n, m: (0, n)),
        ],
        out_specs=(
            pl.BlockSpec((_BM, _BN), lambda n, m: (m, n)),
            pl.BlockSpec((1, 1), lambda n, m: (0, 0)),
        ),
        scratch_shapes=[pltpu.SMEM((1,), jnp.float32)],
        compiler_params=pltpu.CompilerParams(vmem_limit_bytes=60 * 1024 * 1024),
    )(xfull, w)


_EB = 512
_EG = (N_DEV * M_PER) // _EB


def _epi_body(amax_in, y_ref, o_ref, slots, mybuf, send_sems, recv_sems, ack_sem, scale_s):
    step = pl.program_id(0)

    @pl.when(step == 0)
    def _():
        me = lax.axis_index("i")
        barrier_sem = pltpu.get_barrier_semaphore()
        for k in range(1, N_DEV):
            p = lax.rem(me + k, N_DEV)
            pl.semaphore_signal(barrier_sem, inc=1, device_id=(p,), device_id_type=_MESH)
        pl.semaphore_wait(barrier_sem, N_DEV - 1)

        my_amax = amax_in[0, 0]
        mybuf[...] = jnp.full((8, 128), my_amax, jnp.float32)

        sends = []
        for k in range(1, N_DEV):
            p = lax.rem(me + k, N_DEV)
            s = pltpu.make_async_remote_copy(
                src_ref=mybuf,
                dst_ref=slots.at[me],
                send_sem=send_sems.at[k - 1],
                recv_sem=recv_sems.at[me],
                device_id=(p,),
                device_id_type=_MESH,
            )
            s.start()
            sends.append(s)
        for k in range(1, N_DEV):
            p = lax.rem(me + k, N_DEV)
            r = pltpu.make_async_remote_copy(
                src_ref=mybuf,
                dst_ref=slots.at[p],
                send_sem=send_sems.at[k - 1],
                recv_sem=recv_sems.at[p],
                device_id=(p,),
                device_id_type=_MESH,
            )
            r.wait_recv()
        for s in sends:
            s.wait_send()

        vals = slots[:, 0, :]
        row = lax.broadcasted_iota(jnp.int32, (N_DEV, 128), 0)
        g = jnp.max(jnp.where(row == me, -jnp.inf, vals))
        g = jnp.maximum(g, my_amax)
        scale_s[0] = g / F8_MAX
        scale_s[1] = F8_MAX / g

        for k in range(1, N_DEV):
            p = lax.rem(me + k, N_DEV)
            pl.semaphore_signal(ack_sem, inc=1, device_id=(p,), device_id_type=_MESH)
        pl.semaphore_wait(ack_sem, N_DEV - 1)

    scale = scale_s[0]
    inv = scale_s[1]
    q = (y_ref[...] * inv).astype(jnp.float8_e4m3fn).astype(jnp.float32)
    o_ref[...] = q * scale


def _epilogue(y, amax):
    return pl.pallas_call(
        _epi_body,
        grid=(_EG,),
        out_shape=jax.ShapeDtypeStruct((N_DEV * M_PER, N_PER), jnp.float32),
        in_specs=[
            pl.BlockSpec((1, 1), lambda m: (0, 0)),
            pl.BlockSpec((_EB, N_PER), lambda m: (m, 0)),
        ],
        out_specs=pl.BlockSpec((_EB, N_PER), lambda m: (m, 0)),
        scratch_shapes=[
            pltpu.VMEM((N_DEV, 8, 128), jnp.float32),
            pltpu.VMEM((8, 128), jnp.float32),
            pltpu.SemaphoreType.DMA((N_DEV - 1,)),
            pltpu.SemaphoreType.DMA((N_DEV,)),
            pltpu.SemaphoreType.REGULAR,
            pltpu.SMEM((2,), jnp.float32),
        ],
        compiler_params=pltpu.CompilerParams(collective_id=1),
    )(amax, y)


def kernel(x, w_mat):
    xfull = _all_gather(x)
    y, amax = _gemm(xfull, w_mat)
    return _epilogue(y, amax)


# device time: 367590 ns/iter; 1.9193x vs baseline; 1.1893x over previous
import jax
import jax.numpy as jnp
from jax import lax
from jax.experimental import pallas as pl
from jax.experimental.pallas import tpu as pltpu

N_DEV = 4
M_PER = 1024
K = 4096
N_PER = 2048
F8_MAX = 448.0

_MESH = pl.DeviceIdType.MESH


M_HALF = M_PER // 2


_BN = 512
_GN = N_PER // _BN


def _fused_body(x_hbm, w_ref, y_ref, amax_ref, xfull, vstage,
                stage_sems, cw_send, cw_recv, ccw_send, ccw_recv, amax_s):
    c = pl.program_id(0)
    n = pl.program_id(1)
    h = pl.program_id(2)
    me = lax.axis_index("i")
    right = lax.rem(me + 1, N_DEV)
    left = lax.rem(me + N_DEV - 1, N_DEV)
    slot = lax.rem(c, 2)

    def lo_slot(o):
        return xfull.at[pl.ds(o * M_PER, M_HALF), :]

    def hi_slot(o):
        return xfull.at[pl.ds(o * M_PER + M_HALF, M_HALF), :]

    @pl.when(jnp.logical_and(n == 0, h == 0))
    def _comm():
        o_cw = lax.rem(me - c + N_DEV, N_DEV)
        o_ccw = lax.rem(me + c, N_DEV)

        @pl.when(c == 0)
        def _():
            barrier_sem = pltpu.get_barrier_semaphore()
            pl.semaphore_signal(barrier_sem, inc=1, device_id=(left,),
                                device_id_type=_MESH)
            pl.semaphore_signal(barrier_sem, inc=1, device_id=(right,),
                                device_id_type=_MESH)
            pl.semaphore_wait(barrier_sem, 2)
            pltpu.make_async_remote_copy(
                src_ref=x_hbm.at[pl.ds(0, M_HALF), :], dst_ref=lo_slot(me),
                send_sem=cw_send.at[0], recv_sem=cw_recv.at[0],
                device_id=(right,), device_id_type=_MESH,
            ).start()
            pltpu.make_async_remote_copy(
                src_ref=x_hbm.at[pl.ds(M_HALF, M_HALF), :], dst_ref=hi_slot(me),
                send_sem=ccw_send.at[0], recv_sem=ccw_recv.at[0],
                device_id=(left,), device_id_type=_MESH,
            ).start()
            lo_cp = pltpu.make_async_copy(
                x_hbm.at[pl.ds(0, M_HALF), :], vstage.at[slot, 0], stage_sems.at[0])
            hi_cp = pltpu.make_async_copy(
                x_hbm.at[pl.ds(M_HALF, M_HALF), :], vstage.at[slot, 1], stage_sems.at[1])
            lo_cp.start()
            hi_cp.start()
            lo_cp.wait()
            hi_cp.wait()

        @pl.when(c >= 1)
        def _():
            pltpu.make_async_remote_copy(
                src_ref=lo_slot(o_cw), dst_ref=lo_slot(o_cw),
                send_sem=cw_send.at[c - 1], recv_sem=cw_recv.at[c - 1],
                device_id=(right,), device_id_type=_MESH,
            ).wait_recv()
            pltpu.make_async_remote_copy(
                src_ref=hi_slot(o_ccw), dst_ref=hi_slot(o_ccw),
                send_sem=ccw_send.at[c - 1], recv_sem=ccw_recv.at[c - 1],
                device_id=(left,), device_id_type=_MESH,
            ).wait_recv()
            pltpu.make_async_remote_copy(
                src_ref=lo_slot(o_cw), dst_ref=lo_slot(o_cw),
                send_sem=cw_send.at[c - 1], recv_sem=cw_recv.at[c - 1],
                device_id=(right,), device_id_type=_MESH,
            ).wait_send()
            pltpu.make_async_remote_copy(
                src_ref=hi_slot(o_ccw), dst_ref=hi_slot(o_ccw),
                send_sem=ccw_send.at[c - 1], recv_sem=ccw_recv.at[c - 1],
                device_id=(left,), device_id_type=_MESH,
            ).wait_send()

            @pl.when(c <= N_DEV - 2)
            def _():
                pltpu.make_async_remote_copy(
                    src_ref=lo_slot(o_cw), dst_ref=lo_slot(o_cw),
                    send_sem=cw_send.at[c], recv_sem=cw_recv.at[c],
                    device_id=(right,), device_id_type=_MESH,
                ).start()
                pltpu.make_async_remote_copy(
                    src_ref=hi_slot(o_ccw), dst_ref=hi_slot(o_ccw),
                    send_sem=ccw_send.at[c], recv_sem=ccw_recv.at[c],
                    device_id=(left,), device_id_type=_MESH,
                ).start()

            lo_cp = pltpu.make_async_copy(lo_slot(o_cw), vstage.at[slot, 0],
                                          stage_sems.at[0])
            hi_cp = pltpu.make_async_copy(hi_slot(o_ccw), vstage.at[slot, 1],
                                          stage_sems.at[1])
            lo_cp.start()
            hi_cp.start()
            lo_cp.wait()
            hi_cp.wait()

    @pl.when(jnp.logical_and(jnp.logical_and(c == 0, n == 0), h == 0))
    def _():
        amax_s[0] = 0.0

    yb = jnp.maximum(
        jnp.dot(vstage[slot, h], w_ref[...], preferred_element_type=jnp.float32),
        0.0,
    )
    y_ref[...] = yb
    amax_s[0] = jnp.maximum(amax_s[0], jnp.max(yb))

    @pl.when(jnp.logical_and(jnp.logical_and(c == N_DEV - 1, n == _GN - 1), h == 1))
    def _():
        amax_ref[...] = jnp.full((1, 1), amax_s[0], jnp.float32)


def _y_index(c, n, h):
    me = lax.axis_index("i")
    row_lo = 2 * lax.rem(me - c + N_DEV, N_DEV)
    row_hi = 2 * lax.rem(me + c, N_DEV) + 1
    return (jnp.where(h == 0, row_lo, row_hi), n)


def _fused_ag_gemm(x, w):
    y, amax, _ = pl.pallas_call(
        _fused_body,
        grid=(N_DEV, _GN, 2),
        out_shape=(
            jax.ShapeDtypeStruct((N_DEV * M_PER, N_PER), jnp.float32),
            jax.ShapeDtypeStruct((1, 1), jnp.float32),
            jax.ShapeDtypeStruct((N_DEV * M_PER, K), jnp.float32),
        ),
        in_specs=[
            pl.BlockSpec(memory_space=pl.ANY),
            pl.BlockSpec((K, _BN), lambda c, n, h: (0, n)),
        ],
        out_specs=(
            pl.BlockSpec((M_HALF, _BN), _y_index),
            pl.BlockSpec((1, 1), lambda c, n, h: (0, 0)),
            pl.BlockSpec(memory_space=pl.ANY),
        ),
        scratch_shapes=[
            pltpu.VMEM((2, 2, M_HALF, K), jnp.float32),
            pltpu.SemaphoreType.DMA((2,)),
            pltpu.SemaphoreType.DMA((N_DEV - 1,)),
            pltpu.SemaphoreType.DMA((N_DEV - 1,)),
            pltpu.SemaphoreType.DMA((N_DEV - 1,)),
            pltpu.SemaphoreType.DMA((N_DEV - 1,)),
            pltpu.SMEM((1,), jnp.float32),
        ],
        compiler_params=pltpu.CompilerParams(
            collective_id=0, vmem_limit_bytes=60 * 1024 * 1024
        ),
    )(x, w)
    return y, amax


_EB = 512
_EG = (N_DEV * M_PER) // _EB


def _epi_body(amax_in, y_ref, o_ref, slots, mybuf, send_sems, recv_sems, ack_sem, scale_s):
    step = pl.program_id(0)

    @pl.when(step == 0)
    def _():
        me = lax.axis_index("i")
        barrier_sem = pltpu.get_barrier_semaphore()
        for k in range(1, N_DEV):
            p = lax.rem(me + k, N_DEV)
            pl.semaphore_signal(barrier_sem, inc=1, device_id=(p,), device_id_type=_MESH)
        pl.semaphore_wait(barrier_sem, N_DEV - 1)

        my_amax = amax_in[0, 0]
        mybuf[...] = jnp.full((8, 128), my_amax, jnp.float32)

        sends = []
        for k in range(1, N_DEV):
            p = lax.rem(me + k, N_DEV)
            s = pltpu.make_async_remote_copy(
                src_ref=mybuf,
                dst_ref=slots.at[me],
                send_sem=send_sems.at[k - 1],
                recv_sem=recv_sems.at[me],
                device_id=(p,),
                device_id_type=_MESH,
            )
            s.start()
            sends.append(s)
        for k in range(1, N_DEV):
            p = lax.rem(me + k, N_DEV)
            r = pltpu.make_async_remote_copy(
                src_ref=mybuf,
                dst_ref=slots.at[p],
                send_sem=send_sems.at[k - 1],
                recv_sem=recv_sems.at[p],
                device_id=(p,),
                device_id_type=_MESH,
            )
            r.wait_recv()
        for s in sends:
            s.wait_send()

        vals = slots[:, 0, :]
        row = lax.broadcasted_iota(jnp.int32, (N_DEV, 128), 0)
        g = jnp.max(jnp.where(row == me, -jnp.inf, vals))
        g = jnp.maximum(g, my_amax)
        scale_s[0] = g / F8_MAX
        scale_s[1] = F8_MAX / g

        for k in range(1, N_DEV):
            p = lax.rem(me + k, N_DEV)
            pl.semaphore_signal(ack_sem, inc=1, device_id=(p,), device_id_type=_MESH)
        pl.semaphore_wait(ack_sem, N_DEV - 1)

    scale = scale_s[0]
    inv = scale_s[1]
    q = (y_ref[...] * inv).astype(jnp.float8_e4m3fn).astype(jnp.float32)
    o_ref[...] = q * scale


def _epilogue(y, amax):
    return pl.pallas_call(
        _epi_body,
        grid=(_EG,),
        out_shape=jax.ShapeDtypeStruct((N_DEV * M_PER, N_PER), jnp.float32),
        in_specs=[
            pl.BlockSpec((1, 1), lambda m: (0, 0)),
            pl.BlockSpec((_EB, N_PER), lambda m: (m, 0)),
        ],
        out_specs=pl.BlockSpec((_EB, N_PER), lambda m: (m, 0)),
        scratch_shapes=[
            pltpu.VMEM((N_DEV, 8, 128), jnp.float32),
            pltpu.VMEM((8, 128), jnp.float32),
            pltpu.SemaphoreType.DMA((N_DEV - 1,)),
            pltpu.SemaphoreType.DMA((N_DEV,)),
            pltpu.SemaphoreType.REGULAR,
            pltpu.SMEM((2,), jnp.float32),
        ],
        compiler_params=pltpu.CompilerParams(collective_id=1),
    )(amax, y)


def kernel(x, w_mat):
    y, amax = _fused_ag_gemm(x, w_mat)
    return _epilogue(y, amax)
